# Initial kernel scaffold; baseline (speedup 1.0000x reference)
import jax
import jax.numpy as jnp
from jax.experimental import pallas as pl


def kernel(x, edge_index, Wl0, bl0, Wr0, g0, be0, Wl1, bl1, Wr1, g1, be1, Wl2, bl2, Wr2):
    raise NotImplementedError("write your pallas kernel here")


# speedup vs baseline: 3.4914x; 3.4914x over previous
import functools

import jax
import jax.numpy as jnp
from jax import lax
from jax.experimental import pallas as pl
from jax.experimental.pallas import tpu as pltpu
from jax.experimental.pallas import tpu_sc as plsc

N = 10000
E = 160000
DH = 256
DO = 128
EPS = 1e-5

NC = 2
NS = 16
RS = 624

f32 = jnp.float32
i32 = jnp.int32


def _zero_rows(ref, nrows, ncols, value=0.0):
    v16 = jnp.full((16,), value, f32)

    def body(r, _):
        for w in range(ncols // 16):
            ref[r, pl.ds(w * 16, 16)] = v16
        return 0

    lax.fori_loop(0, nrows, body, 0)


def _row_chunks(total, chunk):
    full = total // chunk
    out = [(i * chunk, chunk) for i in range(full)]
    if total - full * chunk:
        out.append((full * chunk, total - full * chunk))
    return out


def _for_sub_rows(s, chunk, fn):
    for off, sz in _row_chunks(RS, chunk):
        fn(s * RS + off, sz)

    @pl.when(s == NS - 1)
    def _():
        fn(NS * RS, N - NS * RS)


def _make_sc_agg(mode):
    col_split = mode == "cols"
    gather = mode != "cnt"
    if col_split:
        epsub = E // NS
        k = 80
    else:
        epsub = E // (NC * NS)
        k = 40
    nch = epsub // k
    assert nch * k == epsub and k % 8 == 0 and k <= 128

    scratch = [
        pltpu.VMEM((k,), i32),
        pltpu.VMEM((k,), i32),
        pltpu.VMEM((k, 128), f32),
        pltpu.VMEM_SHARED((N, 128), f32),
        pltpu.SemaphoreType.DMA,
    ]

    mesh = plsc.VectorSubcoreMesh(core_axis_name="c", subcore_axis_name="s",
                                  num_cores=NC, num_subcores=NS)

    def body(tbl_hbm, src_hbm, dst_hbm, agg_hbm, src_v, dst_v, rows_v, acc,
             sem):
        c = lax.axis_index("c")
        s = lax.axis_index("s")

        _zero_rows(rows_v, k, 128)
        _for_sub_rows(s, k, lambda off, sz: pltpu.sync_copy(
            rows_v.at[pl.ds(0, sz)], acc.at[pl.ds(off, sz)]))
        plsc.subcore_barrier()

        if col_split:
            ebase = s * epsub
        else:
            ebase = c * (E // NC) + s * epsub
        if not gather:
            _zero_rows(rows_v, k, 128, value=1.0)

        def edge_body(ic, _):
            base = ebase + ic * k
            pltpu.sync_copy(dst_hbm.at[pl.ds(base, k)], dst_v)
            if gather:
                pltpu.sync_copy(src_hbm.at[pl.ds(base, k)], src_v)
                if col_split:
                    for j in range(k // 16):
                        sl = pl.ds(j * 16, 16)
                        src_v[sl] = src_v[sl] + c * N
                pltpu.async_copy(tbl_hbm.at[src_v], rows_v, sem).wait()
            pltpu.sync_copy(rows_v, acc.at[dst_v], add=True)
            return 0

        lax.fori_loop(0, nch, edge_body, 0)
        plsc.subcore_barrier()

        def wb(off, sz):
            pltpu.sync_copy(acc.at[pl.ds(off, sz)], rows_v.at[pl.ds(0, sz)])
            pltpu.sync_copy(rows_v.at[pl.ds(0, sz)],
                            agg_hbm.at[pl.ds(c * N + off, sz)])

        _for_sub_rows(s, k, wb)

    if not gather:
        def body_nogather(src_hbm, dst_hbm, agg_hbm, *rest):
            return body(None, src_hbm, dst_hbm, agg_hbm, *rest)
        fn = body_nogather
    else:
        fn = body
    return pl.kernel(fn, out_type=jax.ShapeDtypeStruct((2 * N, 128), f32),
                     mesh=mesh, scratch_types=scratch)


_make_sc_agg = functools.lru_cache(maxsize=None)(_make_sc_agg)


def _sc_cols(tbl, src, dst):
    return _make_sc_agg("cols")(tbl, src, dst)


def _sc_rows(tbl, src, dst):
    return _make_sc_agg("rows")(tbl, src, dst)


def _sc_cnt(src, dst):
    return _make_sc_agg("cnt")(src, dst)


BR = 1000
NB = N // BR


def _tc_pre_body(x_ref, wl_ref, wr_ref, bl_ref, yl_ref, yr_ref):
    xb = x_ref[...]
    yl = lax.dot_general(xb, wl_ref[...], (((1,), (0,)), ((), ())),
                         preferred_element_type=f32)
    yl_ref[0] = yl[:, :128]
    yl_ref[1] = yl[:, 128:]
    yr_ref[...] = lax.dot_general(xb, wr_ref[...], (((1,), (0,)), ((), ())),
                                  preferred_element_type=f32) + bl_ref[0]


def _tc_pre(x, wl, wr, bl):
    return pl.pallas_call(
        _tc_pre_body,
        grid=(NB,),
        in_specs=[
            pl.BlockSpec((BR, DH), lambda i: (i, 0)),
            pl.BlockSpec((DH, DH), lambda i: (0, 0)),
            pl.BlockSpec((DH, DH), lambda i: (0, 0)),
            pl.BlockSpec((1, DH), lambda i: (0, 0)),
        ],
        out_specs=[
            pl.BlockSpec((2, BR, 128), lambda i: (0, i, 0)),
            pl.BlockSpec((BR, DH), lambda i: (i, 0)),
        ],
        out_shape=[
            jax.ShapeDtypeStruct((2, N, 128), f32),
            jax.ShapeDtypeStruct((N, DH), f32),
        ],
    )(x, wl, wr, bl[None, :])


def _tc_mid_body(split_out, dout,
                 agg_ref, cnt_ref, yr_ref, g_ref, be_ref, wl_ref, wr_ref,
                 bl_ref, yl_ref, yr2_ref, stats):
    p = pl.program_id(0)
    i = pl.program_id(1)
    agg = jnp.concatenate([agg_ref[0], agg_ref[1]], axis=1)
    cnt = jnp.maximum(cnt_ref[0] + cnt_ref[1], 1.0)
    h = agg / cnt + yr_ref[...]

    @pl.when(p == 0)
    def _():
        @pl.when(i == 0)
        def _():
            stats[...] = jnp.zeros_like(stats)
        stats[0:1, :] += jnp.sum(h, axis=0, keepdims=True)
        stats[1:2, :] += jnp.sum(h * h, axis=0, keepdims=True)

    @pl.when(p == 1)
    def _():
        mu = stats[0:1, :] / N
        var = stats[1:2, :] / N - mu * mu
        hn = (h - mu) * lax.rsqrt(var + EPS) * g_ref[0] + be_ref[0]
        hn = jnp.maximum(hn, 0.0)
        yl = lax.dot_general(hn, wl_ref[...], (((1,), (0,)), ((), ())),
                             preferred_element_type=f32)
        if split_out:
            yl_ref[0] = yl[:, :dout // 2]
            yl_ref[1] = yl[:, dout // 2:]
        else:
            yl_ref[...] = yl
        yr2_ref[...] = lax.dot_general(hn, wr_ref[...], (((1,), (0,)), ((), ())),
                                       preferred_element_type=f32) + bl_ref[0]


def _tc_mid(agg, cntp, yr, g, be, wl, wr, bl, split_out):
    dout = wl.shape[1]
    if split_out:
        yl_spec = pl.BlockSpec((2, BR, dout // 2), lambda p, i: (0, i, 0))
        yl_shape = jax.ShapeDtypeStruct((2, N, dout // 2), f32)
    else:
        yl_spec = pl.BlockSpec((BR, dout), lambda p, i: (i, 0))
        yl_shape = jax.ShapeDtypeStruct((N, dout), f32)
    return pl.pallas_call(
        functools.partial(_tc_mid_body, split_out, dout),
        grid=(2, NB),
        in_specs=[
            pl.BlockSpec((2, BR, 128), lambda p, i: (0, i, 0)),
            pl.BlockSpec((2, BR, 1), lambda p, i: (0, i, 0)),
            pl.BlockSpec((BR, DH), lambda p, i: (i, 0)),
            pl.BlockSpec((1, DH), lambda p, i: (0, 0)),
            pl.BlockSpec((1, DH), lambda p, i: (0, 0)),
            pl.BlockSpec((DH, dout), lambda p, i: (0, 0)),
            pl.BlockSpec((DH, dout), lambda p, i: (0, 0)),
            pl.BlockSpec((1, dout), lambda p, i: (0, 0)),
        ],
        out_specs=[
            yl_spec,
            pl.BlockSpec((BR, dout), lambda p, i: (i, 0)),
        ],
        out_shape=[
            yl_shape,
            jax.ShapeDtypeStruct((N, dout), f32),
        ],
        scratch_shapes=[pltpu.VMEM((8, DH), f32)],
    )(agg, cntp, yr, g[None, :], be[None, :], wl, wr, bl[None, :])


def _tc_post_body(part_ref, cnt_ref, yr_ref, out_ref):
    agg = part_ref[0] + part_ref[1]
    cnt = jnp.maximum(cnt_ref[0] + cnt_ref[1], 1.0)
    out_ref[...] = agg / cnt + yr_ref[...]


def _tc_post(part, cntp, yr):
    return pl.pallas_call(
        _tc_post_body,
        grid=(NB,),
        in_specs=[
            pl.BlockSpec((2, BR, DO), lambda i: (0, i, 0)),
            pl.BlockSpec((2, BR, 1), lambda i: (0, i, 0)),
            pl.BlockSpec((BR, DO), lambda i: (i, 0)),
        ],
        out_specs=pl.BlockSpec((BR, DO), lambda i: (i, 0)),
        out_shape=jax.ShapeDtypeStruct((N, DO), f32),
    )(part, cntp, yr)


def kernel(x, edge_index, Wl0, bl0, Wr0, g0, be0, Wl1, bl1, Wr1, g1, be1,
           Wl2, bl2, Wr2):
    src = edge_index[0]
    dst = edge_index[1]

    cntp = _sc_cnt(src, dst).reshape(2, N, 128)[:, :, 0:1]
    yl0, yr0 = _tc_pre(x, Wl0, Wr0, bl0)
    agg0 = _sc_cols(yl0.reshape(2 * N, 128), src, dst)
    yl1, yr1 = _tc_mid(agg0.reshape(2, N, 128), cntp, yr0, g0, be0,
                       Wl1, Wr1, bl1, split_out=True)
    agg1 = _sc_cols(yl1.reshape(2 * N, 128), src, dst)
    yl2, yr2 = _tc_mid(agg1.reshape(2, N, 128), cntp, yr1, g1, be1,
                       Wl2, Wr2, bl2, split_out=False)
    part2 = _sc_rows(yl2, src, dst)
    return _tc_post(part2.reshape(2, N, DO), cntp, yr2)


# speedup vs baseline: 6.3246x; 1.8115x over previous
import functools

import jax
import jax.numpy as jnp
from jax import lax
from jax.experimental import pallas as pl
from jax.experimental.pallas import tpu as pltpu
from jax.experimental.pallas import tpu_sc as plsc

N = 10000
E = 160000
DH = 256
DO = 128
EPS = 1e-5

NC = 2
NS = 16
RS = 624

f32 = jnp.float32
i32 = jnp.int32


def _zero_rows(ref, nrows, ncols, value=0.0):
    v16 = jnp.full((16,), value, f32)

    def body(r, _):
        for w in range(ncols // 16):
            ref[r, pl.ds(w * 16, 16)] = v16
        return 0

    lax.fori_loop(0, nrows, body, 0)


def _row_chunks(total, chunk):
    full = total // chunk
    out = [(i * chunk, chunk) for i in range(full)]
    if total - full * chunk:
        out.append((full * chunk, total - full * chunk))
    return out


def _for_sub_rows(s, chunk, fn):
    for off, sz in _row_chunks(RS, chunk):
        fn(s * RS + off, sz)

    @pl.when(s == NS - 1)
    def _():
        fn(NS * RS, N - NS * RS)


def _make_sc_agg(mode):
    col_split = mode == "cols"
    gather = mode != "cnt"
    K = 125
    if col_split:
        nch = (E // NS) // K
    else:
        nch = (E // (NC * NS)) // K
    WBK = 80

    scratch = [
        pltpu.VMEM((nch, K), i32),
        pltpu.VMEM((K, 128), f32),
        pltpu.VMEM_SHARED((N, 128), f32),
        pltpu.SemaphoreType.DMA,
    ]
    if gather:
        scratch += [
            pltpu.VMEM((nch, K), i32),
            pltpu.VMEM((K, 128), f32),
            pltpu.SemaphoreType.DMA,
        ]

    mesh = plsc.VectorSubcoreMesh(core_axis_name="c", subcore_axis_name="s",
                                  num_cores=NC, num_subcores=NS)

    def body(tbl_hbm, src_hbm, dst_hbm, agg_hbm, dbig, rows0, acc, ssem,
             *rest):
        if gather:
            sbig, rows1, gsem = rest
            rows = [rows0, rows1]
        else:
            rows = [rows0, rows0]
        c = lax.axis_index("c")
        s = lax.axis_index("s")

        if col_split:
            drow = s * nch
            srow = c * (E // K) + s * nch
        else:
            drow = c * (E // (NC * K)) + s * nch
            srow = drow

        pltpu.sync_copy(dst_hbm.at[pl.ds(drow, nch)], dbig)
        if gather:
            pltpu.sync_copy(src_hbm.at[pl.ds(srow, nch)], sbig)

        _zero_rows(rows0, WBK, 128)
        _for_sub_rows(s, WBK, lambda off, sz: pltpu.sync_copy(
            rows0.at[pl.ds(0, sz)], acc.at[pl.ds(off, sz)]))
        if not gather:
            _zero_rows(rows0, K, 128, value=1.0)
        plsc.subcore_barrier()

        def issue_g(b, ic):
            return pltpu.async_copy(tbl_hbm.at[sbig.at[ic]], rows[b], gsem)

        def issue_s(b, ic):
            return pltpu.async_copy(rows[b], acc.at[dbig.at[ic]], ssem,
                                    add=True)

        if gather:
            def edge_body(ic, _):
                issue_g(0, ic).wait()
                issue_s(0, ic).wait()
                return 0
        else:
            def edge_body(ic, _):
                issue_s(0, ic).wait()
                return 0

        lax.fori_loop(0, nch, edge_body, 0)
        plsc.subcore_barrier()

        def wb(off, sz):
            pltpu.sync_copy(acc.at[pl.ds(off, sz)], rows0.at[pl.ds(0, sz)])
            pltpu.sync_copy(rows0.at[pl.ds(0, sz)],
                            agg_hbm.at[pl.ds(c * N + off, sz)])

        _for_sub_rows(s, WBK, wb)

    if not gather:
        def body_nogather(src_hbm, dst_hbm, agg_hbm, *rest):
            return body(None, src_hbm, dst_hbm, agg_hbm, *rest)
        fn = body_nogather
    else:
        fn = body
    return pl.kernel(fn, out_type=jax.ShapeDtypeStruct((2 * N, 128), f32),
                     mesh=mesh, scratch_types=scratch)


_make_sc_agg = functools.lru_cache(maxsize=None)(_make_sc_agg)


def _sc_cols(tbl, src, dst):
    return _make_sc_agg("cols")(tbl, src, dst)


def _sc_rows(tbl, src, dst):
    return _make_sc_agg("rows")(tbl, src, dst)


def _sc_cnt(src, dst):
    return _make_sc_agg("cnt")(src, dst)


BR = 1000
NB = N // BR


def _tc_pre_body(x_ref, wl_ref, wr_ref, bl_ref, yl_ref, yr_ref):
    xb = x_ref[...]
    yl = lax.dot_general(xb, wl_ref[...], (((1,), (0,)), ((), ())),
                         preferred_element_type=f32)
    yl_ref[0] = yl[:, :128]
    yl_ref[1] = yl[:, 128:]
    yr_ref[...] = lax.dot_general(xb, wr_ref[...], (((1,), (0,)), ((), ())),
                                  preferred_element_type=f32) + bl_ref[0]


def _tc_pre(x, wl, wr, bl):
    return pl.pallas_call(
        _tc_pre_body,
        grid=(NB,),
        in_specs=[
            pl.BlockSpec((BR, DH), lambda i: (i, 0)),
            pl.BlockSpec((DH, DH), lambda i: (0, 0)),
            pl.BlockSpec((DH, DH), lambda i: (0, 0)),
            pl.BlockSpec((1, DH), lambda i: (0, 0)),
        ],
        out_specs=[
            pl.BlockSpec((2, BR, 128), lambda i: (0, i, 0)),
            pl.BlockSpec((BR, DH), lambda i: (i, 0)),
        ],
        out_shape=[
            jax.ShapeDtypeStruct((2, N, 128), f32),
            jax.ShapeDtypeStruct((N, DH), f32),
        ],
    )(x, wl, wr, bl[None, :])


def _tc_mid_body(split_out, dout,
                 agg_ref, cnt_ref, yr_ref, g_ref, be_ref, wl_ref, wr_ref,
                 bl_ref, yl_ref, yr2_ref, stats):
    p = pl.program_id(0)
    i = pl.program_id(1)
    agg = jnp.concatenate([agg_ref[0], agg_ref[1]], axis=1)
    cnt = jnp.maximum(cnt_ref[0] + cnt_ref[1], 1.0)
    h = agg / cnt + yr_ref[...]

    @pl.when(p == 0)
    def _():
        @pl.when(i == 0)
        def _():
            stats[...] = jnp.zeros_like(stats)
        stats[0:1, :] += jnp.sum(h, axis=0, keepdims=True)
        stats[1:2, :] += jnp.sum(h * h, axis=0, keepdims=True)

    @pl.when(p == 1)
    def _():
        mu = stats[0:1, :] / N
        var = stats[1:2, :] / N - mu * mu
        hn = (h - mu) * lax.rsqrt(var + EPS) * g_ref[0] + be_ref[0]
        hn = jnp.maximum(hn, 0.0)
        yl = lax.dot_general(hn, wl_ref[...], (((1,), (0,)), ((), ())),
                             preferred_element_type=f32)
        if split_out:
            yl_ref[0] = yl[:, :dout // 2]
            yl_ref[1] = yl[:, dout // 2:]
        else:
            yl_ref[...] = yl
        yr2_ref[...] = lax.dot_general(hn, wr_ref[...], (((1,), (0,)), ((), ())),
                                       preferred_element_type=f32) + bl_ref[0]


def _tc_mid(agg, cntp, yr, g, be, wl, wr, bl, split_out):
    dout = wl.shape[1]
    if split_out:
        yl_spec = pl.BlockSpec((2, BR, dout // 2), lambda p, i: (0, i, 0))
        yl_shape = jax.ShapeDtypeStruct((2, N, dout // 2), f32)
    else:
        yl_spec = pl.BlockSpec((BR, dout), lambda p, i: (i, 0))
        yl_shape = jax.ShapeDtypeStruct((N, dout), f32)
    return pl.pallas_call(
        functools.partial(_tc_mid_body, split_out, dout),
        grid=(2, NB),
        in_specs=[
            pl.BlockSpec((2, BR, 128), lambda p, i: (0, i, 0)),
            pl.BlockSpec((2, BR, 1), lambda p, i: (0, i, 0)),
            pl.BlockSpec((BR, DH), lambda p, i: (i, 0)),
            pl.BlockSpec((1, DH), lambda p, i: (0, 0)),
            pl.BlockSpec((1, DH), lambda p, i: (0, 0)),
            pl.BlockSpec((DH, dout), lambda p, i: (0, 0)),
            pl.BlockSpec((DH, dout), lambda p, i: (0, 0)),
            pl.BlockSpec((1, dout), lambda p, i: (0, 0)),
        ],
        out_specs=[
            yl_spec,
            pl.BlockSpec((BR, dout), lambda p, i: (i, 0)),
        ],
        out_shape=[
            yl_shape,
            jax.ShapeDtypeStruct((N, dout), f32),
        ],
        scratch_shapes=[pltpu.VMEM((8, DH), f32)],
    )(agg, cntp, yr, g[None, :], be[None, :], wl, wr, bl[None, :])


def _tc_post_body(part_ref, cnt_ref, yr_ref, out_ref):
    agg = part_ref[0] + part_ref[1]
    cnt = jnp.maximum(cnt_ref[0] + cnt_ref[1], 1.0)
    out_ref[...] = agg / cnt + yr_ref[...]


def _tc_post(part, cntp, yr):
    return pl.pallas_call(
        _tc_post_body,
        grid=(NB,),
        in_specs=[
            pl.BlockSpec((2, BR, DO), lambda i: (0, i, 0)),
            pl.BlockSpec((2, BR, 1), lambda i: (0, i, 0)),
            pl.BlockSpec((BR, DO), lambda i: (i, 0)),
        ],
        out_specs=pl.BlockSpec((BR, DO), lambda i: (i, 0)),
        out_shape=jax.ShapeDtypeStruct((N, DO), f32),
    )(part, cntp, yr)


def kernel(x, edge_index, Wl0, bl0, Wr0, g0, be0, Wl1, bl1, Wr1, g1, be1,
           Wl2, bl2, Wr2):
    src = edge_index[0]
    dst = edge_index[1]
    K = 125
    src2 = jnp.concatenate([src, src + N]).reshape(2 * E // K, K)
    srcr = src.reshape(E // K, K)
    dstr = dst.reshape(E // K, K)

    cntp = _sc_cnt(srcr, dstr).reshape(2, N, 128)[:, :, 0:1]
    yl0, yr0 = _tc_pre(x, Wl0, Wr0, bl0)
    agg0 = _sc_cols(yl0.reshape(2 * N, 128), src2, dstr)
    yl1, yr1 = _tc_mid(agg0.reshape(2, N, 128), cntp, yr0, g0, be0,
                       Wl1, Wr1, bl1, split_out=True)
    agg1 = _sc_cols(yl1.reshape(2 * N, 128), src2, dstr)
    yl2, yr2 = _tc_mid(agg1.reshape(2, N, 128), cntp, yr1, g1, be1,
                       Wl2, Wr2, bl2, split_out=False)
    part2 = _sc_rows(yl2, srcr, dstr)
    return _tc_post(part2.reshape(2, N, DO), cntp, yr2)


# speedup vs baseline: 7.7773x; 1.2297x over previous
import functools

import jax
import jax.numpy as jnp
from jax import lax
from jax.experimental import pallas as pl
from jax.experimental.pallas import tpu as pltpu
from jax.experimental.pallas import tpu_sc as plsc

N = 10000
E = 160000
DH = 256
DO = 128
EPS = 1e-5

NC = 2
NS = 16
RS = 624

f32 = jnp.float32
i32 = jnp.int32


def _zero_rows(ref, nrows, ncols, value=0.0):
    v16 = jnp.full((16,), value, f32)

    def body(r, _):
        for w in range(ncols // 16):
            ref[r, pl.ds(w * 16, 16)] = v16
        return 0

    lax.fori_loop(0, nrows, body, 0)


def _row_chunks(total, chunk):
    full = total // chunk
    out = [(i * chunk, chunk) for i in range(full)]
    if total - full * chunk:
        out.append((full * chunk, total - full * chunk))
    return out


def _for_sub_rows(s, chunk, fn):
    for off, sz in _row_chunks(RS, chunk):
        fn(s * RS + off, sz)

    @pl.when(s == NS - 1)
    def _():
        fn(NS * RS, N - NS * RS)


def _make_sc_agg(mode):
    col_split = mode == "cols"
    gather = mode != "cnt"
    K = 125
    if col_split:
        nch = (E // NS) // K
    else:
        nch = (E // (NC * NS)) // K
    WBK = 80

    nhalf = 2 if col_split else 1
    nst = nch // nhalf
    scratch = [
        pltpu.VMEM((nst, K), i32),
        pltpu.VMEM((K, 128), f32),
        pltpu.VMEM_SHARED((N, 128), f32),
        pltpu.SemaphoreType.DMA,
    ]
    if gather:
        scratch += [
            pltpu.VMEM((nst, K), i32),
            pltpu.VMEM((K, 128), f32),
            pltpu.SemaphoreType.DMA,
        ]

    mesh = plsc.VectorSubcoreMesh(core_axis_name="c", subcore_axis_name="s",
                                  num_cores=NC, num_subcores=NS)

    def body(tbl_hbm, src_hbm, dst_hbm, agg_hbm, dbig, rows0, acc, ssem,
             *rest):
        if gather:
            sbig, rows1, gsem = rest
            rows = [rows0, rows1]
        else:
            rows = [rows0, rows0]
        c = lax.axis_index("c")
        s = lax.axis_index("s")

        if col_split:
            drow = s * nch
            srow = c * (E // K) + s * nch
        else:
            drow = c * (E // (NC * K)) + s * nch
            srow = drow

        pltpu.sync_copy(dst_hbm.at[pl.ds(drow, nst)], dbig)
        if gather:
            pltpu.sync_copy(src_hbm.at[pl.ds(srow, nst)], sbig)

        _zero_rows(rows0, WBK, 128)
        _for_sub_rows(s, WBK, lambda off, sz: pltpu.sync_copy(
            rows0.at[pl.ds(0, sz)], acc.at[pl.ds(off, sz)]))
        if not gather:
            _zero_rows(rows0, K, 128, value=1.0)
        plsc.subcore_barrier()

        def issue_g(b, ic):
            return pltpu.async_copy(tbl_hbm.at[sbig.at[ic]], rows[b], gsem)

        def issue_s(b, ic):
            return pltpu.async_copy(rows[b], acc.at[dbig.at[ic]], ssem,
                                    add=True)

        if gather:
            def edge_body(ip, _):
                ic = ip * 2
                g0 = issue_g(0, ic)
                g1 = issue_g(1, ic + 1)
                g0.wait()
                s0 = issue_s(0, ic)
                g1.wait()
                s1 = issue_s(1, ic + 1)
                s0.wait()
                s1.wait()
                return 0
        else:
            def edge_body(ip, _):
                ic = ip * 2
                s0 = issue_s(0, ic)
                s1 = issue_s(1, ic + 1)
                s0.wait()
                s1.wait()
                return 0

        assert nst % 2 == 0
        for half in range(nhalf):
            if half > 0:
                pltpu.sync_copy(dst_hbm.at[pl.ds(drow + half * nst, nst)],
                                dbig)
                if gather:
                    pltpu.sync_copy(src_hbm.at[pl.ds(srow + half * nst, nst)],
                                    sbig)
            lax.fori_loop(0, nst // 2, edge_body, 0)
        plsc.subcore_barrier()

        def wb(off, sz):
            pltpu.sync_copy(acc.at[pl.ds(off, sz)], rows0.at[pl.ds(0, sz)])
            pltpu.sync_copy(rows0.at[pl.ds(0, sz)],
                            agg_hbm.at[pl.ds(c * N + off, sz)])

        _for_sub_rows(s, WBK, wb)

    if not gather:
        def body_nogather(src_hbm, dst_hbm, agg_hbm, *rest):
            return body(None, src_hbm, dst_hbm, agg_hbm, *rest)
        fn = body_nogather
    else:
        fn = body
    return pl.kernel(fn, out_type=jax.ShapeDtypeStruct((2 * N, 128), f32),
                     mesh=mesh, scratch_types=scratch)


_make_sc_agg = functools.lru_cache(maxsize=None)(_make_sc_agg)


def _sc_cols(tbl, src, dst):
    return _make_sc_agg("cols")(tbl, src, dst)


def _sc_rows(tbl, src, dst):
    return _make_sc_agg("rows")(tbl, src, dst)


def _sc_cnt(src, dst):
    return _make_sc_agg("cnt")(src, dst)


BR = 1000
NB = N // BR


def _tc_pre_body(x_ref, wl_ref, wr_ref, bl_ref, yl_ref, yr_ref):
    xb = x_ref[...]
    yl = lax.dot_general(xb, wl_ref[...], (((1,), (0,)), ((), ())),
                         preferred_element_type=f32)
    yl_ref[0] = yl[:, :128]
    yl_ref[1] = yl[:, 128:]
    yr_ref[...] = lax.dot_general(xb, wr_ref[...], (((1,), (0,)), ((), ())),
                                  preferred_element_type=f32) + bl_ref[0]


def _tc_pre(x, wl, wr, bl):
    return pl.pallas_call(
        _tc_pre_body,
        grid=(NB,),
        in_specs=[
            pl.BlockSpec((BR, DH), lambda i: (i, 0)),
            pl.BlockSpec((DH, DH), lambda i: (0, 0)),
            pl.BlockSpec((DH, DH), lambda i: (0, 0)),
            pl.BlockSpec((1, DH), lambda i: (0, 0)),
        ],
        out_specs=[
            pl.BlockSpec((2, BR, 128), lambda i: (0, i, 0)),
            pl.BlockSpec((BR, DH), lambda i: (i, 0)),
        ],
        out_shape=[
            jax.ShapeDtypeStruct((2, N, 128), f32),
            jax.ShapeDtypeStruct((N, DH), f32),
        ],
    )(x, wl, wr, bl[None, :])


def _tc_mid_body(split_out, dout,
                 agg_ref, cnt_ref, yr_ref, g_ref, be_ref, wl_ref, wr_ref,
                 bl_ref, yl_ref, yr2_ref, stats):
    p = pl.program_id(0)
    i = pl.program_id(1)
    agg = jnp.concatenate([agg_ref[0], agg_ref[1]], axis=1)
    cnt = jnp.maximum(cnt_ref[0] + cnt_ref[1], 1.0)
    h = agg / cnt + yr_ref[...]

    @pl.when(p == 0)
    def _():
        @pl.when(i == 0)
        def _():
            stats[...] = jnp.zeros_like(stats)
        stats[0:1, :] += jnp.sum(h, axis=0, keepdims=True)
        stats[1:2, :] += jnp.sum(h * h, axis=0, keepdims=True)

    @pl.when(p == 1)
    def _():
        mu = stats[0:1, :] / N
        var = stats[1:2, :] / N - mu * mu
        hn = (h - mu) * lax.rsqrt(var + EPS) * g_ref[0] + be_ref[0]
        hn = jnp.maximum(hn, 0.0)
        yl = lax.dot_general(hn, wl_ref[...], (((1,), (0,)), ((), ())),
                             preferred_element_type=f32)
        if split_out:
            yl_ref[0] = yl[:, :dout // 2]
            yl_ref[1] = yl[:, dout // 2:]
        else:
            yl_ref[...] = yl
        yr2_ref[...] = lax.dot_general(hn, wr_ref[...], (((1,), (0,)), ((), ())),
                                       preferred_element_type=f32) + bl_ref[0]


def _tc_mid(agg, cntp, yr, g, be, wl, wr, bl, split_out):
    dout = wl.shape[1]
    if split_out:
        yl_spec = pl.BlockSpec((2, BR, dout // 2), lambda p, i: (0, i, 0))
        yl_shape = jax.ShapeDtypeStruct((2, N, dout // 2), f32)
    else:
        yl_spec = pl.BlockSpec((BR, dout), lambda p, i: (i, 0))
        yl_shape = jax.ShapeDtypeStruct((N, dout), f32)
    return pl.pallas_call(
        functools.partial(_tc_mid_body, split_out, dout),
        grid=(2, NB),
        in_specs=[
            pl.BlockSpec((2, BR, 128), lambda p, i: (0, i, 0)),
            pl.BlockSpec((2, BR, 1), lambda p, i: (0, i, 0)),
            pl.BlockSpec((BR, DH), lambda p, i: (i, 0)),
            pl.BlockSpec((1, DH), lambda p, i: (0, 0)),
            pl.BlockSpec((1, DH), lambda p, i: (0, 0)),
            pl.BlockSpec((DH, dout), lambda p, i: (0, 0)),
            pl.BlockSpec((DH, dout), lambda p, i: (0, 0)),
            pl.BlockSpec((1, dout), lambda p, i: (0, 0)),
        ],
        out_specs=[
            yl_spec,
            pl.BlockSpec((BR, dout), lambda p, i: (i, 0)),
        ],
        out_shape=[
            yl_shape,
            jax.ShapeDtypeStruct((N, dout), f32),
        ],
        scratch_shapes=[pltpu.VMEM((8, DH), f32)],
    )(agg, cntp, yr, g[None, :], be[None, :], wl, wr, bl[None, :])


def _tc_post_body(part_ref, cnt_ref, yr_ref, out_ref):
    agg = part_ref[0] + part_ref[1]
    cnt = jnp.maximum(cnt_ref[0] + cnt_ref[1], 1.0)
    out_ref[...] = agg / cnt + yr_ref[...]


def _tc_post(part, cntp, yr):
    return pl.pallas_call(
        _tc_post_body,
        grid=(NB,),
        in_specs=[
            pl.BlockSpec((2, BR, DO), lambda i: (0, i, 0)),
            pl.BlockSpec((2, BR, 1), lambda i: (0, i, 0)),
            pl.BlockSpec((BR, DO), lambda i: (i, 0)),
        ],
        out_specs=pl.BlockSpec((BR, DO), lambda i: (i, 0)),
        out_shape=jax.ShapeDtypeStruct((N, DO), f32),
    )(part, cntp, yr)


def kernel(x, edge_index, Wl0, bl0, Wr0, g0, be0, Wl1, bl1, Wr1, g1, be1,
           Wl2, bl2, Wr2):
    src = edge_index[0]
    dst = edge_index[1]
    K = 125
    src2 = jnp.concatenate([src, src + N]).reshape(2 * E // K, K)
    srcr = src.reshape(E // K, K)
    dstr = dst.reshape(E // K, K)

    cntp = _sc_cnt(srcr, dstr).reshape(2, N, 128)[:, :, 0:1]
    yl0, yr0 = _tc_pre(x, Wl0, Wr0, bl0)
    agg0 = _sc_cols(yl0.reshape(2 * N, 128), src2, dstr)
    yl1, yr1 = _tc_mid(agg0.reshape(2, N, 128), cntp, yr0, g0, be0,
                       Wl1, Wr1, bl1, split_out=True)
    agg1 = _sc_cols(yl1.reshape(2 * N, 128), src2, dstr)
    yl2, yr2 = _tc_mid(agg1.reshape(2, N, 128), cntp, yr1, g1, be1,
                       Wl2, Wr2, bl2, split_out=False)
    part2 = _sc_rows(yl2, srcr, dstr)
    return _tc_post(part2.reshape(2, N, DO), cntp, yr2)


# speedup vs baseline: 7.8121x; 1.0045x over previous
import functools

import jax
import jax.numpy as jnp
from jax import lax
from jax.experimental import pallas as pl
from jax.experimental.pallas import tpu as pltpu
from jax.experimental.pallas import tpu_sc as plsc

N = 10000
E = 160000
DH = 256
DO = 128
EPS = 1e-5

NC = 2
NS = 16
RS = 624

f32 = jnp.float32
i32 = jnp.int32


def _zero_rows(ref, nrows, ncols, value=0.0):
    v16 = jnp.full((16,), value, f32)

    def body(r, _):
        for w in range(ncols // 16):
            ref[r, pl.ds(w * 16, 16)] = v16
        return 0

    lax.fori_loop(0, nrows, body, 0)


def _row_chunks(total, chunk):
    full = total // chunk
    out = [(i * chunk, chunk) for i in range(full)]
    if total - full * chunk:
        out.append((full * chunk, total - full * chunk))
    return out


def _for_sub_rows(s, chunk, fn):
    for off, sz in _row_chunks(RS, chunk):
        fn(s * RS + off, sz)

    @pl.when(s == NS - 1)
    def _():
        fn(NS * RS, N - NS * RS)


def _make_sc_agg(mode):
    col_split = mode == "cols"
    gather = mode != "cnt"
    K = 125
    if col_split:
        nch = (E // NS) // K
    else:
        nch = (E // (NC * NS)) // K
    WBK = 80

    nhalf = 2 if col_split else 1
    nst = nch // nhalf
    scratch = [
        pltpu.VMEM((nst, K), i32),
        pltpu.VMEM((K, 128), f32),
        pltpu.VMEM_SHARED((N, 128), f32),
        pltpu.SemaphoreType.DMA,
    ]
    if gather:
        scratch += [
            pltpu.VMEM((nst, K), i32),
            pltpu.VMEM((K, 128), f32),
            pltpu.SemaphoreType.DMA,
        ]

    mesh = plsc.VectorSubcoreMesh(core_axis_name="c", subcore_axis_name="s",
                                  num_cores=NC, num_subcores=NS)

    def body(tbl_hbm, src_hbm, dst_hbm, agg_hbm, dbig, rows0, acc, ssem,
             *rest):
        if gather:
            sbig, rows1, gsem = rest
            rows = [rows0, rows1]
        else:
            rows = [rows0, rows0]
        c = lax.axis_index("c")
        s = lax.axis_index("s")

        if col_split:
            drow = s * nch
            srow = c * (E // K) + s * nch
        else:
            drow = c * (E // (NC * K)) + s * nch
            srow = drow

        pltpu.sync_copy(dst_hbm.at[pl.ds(drow, nst)], dbig)
        if gather:
            pltpu.sync_copy(src_hbm.at[pl.ds(srow, nst)], sbig)

        _zero_rows(rows0, WBK, 128)
        _for_sub_rows(s, WBK, lambda off, sz: pltpu.sync_copy(
            rows0.at[pl.ds(0, sz)], acc.at[pl.ds(off, sz)]))
        if not gather:
            _zero_rows(rows0, K, 128, value=1.0)
        plsc.subcore_barrier()

        def issue_g(b, ic):
            return pltpu.async_copy(tbl_hbm.at[sbig.at[ic]], rows[b], gsem)

        def issue_s(b, ic):
            return pltpu.async_copy(rows[b], acc.at[dbig.at[ic]], ssem,
                                    add=True)

        if gather:
            def edge_body(ip, _):
                ic = ip * 2
                g0 = issue_g(0, ic)
                g1 = issue_g(1, ic + 1)
                g0.wait()
                s0 = issue_s(0, ic)
                g1.wait()
                s1 = issue_s(1, ic + 1)
                s0.wait()
                s1.wait()
                return 0
        else:
            def edge_body(ip, _):
                ic = ip * 2
                s0 = issue_s(0, ic)
                s1 = issue_s(1, ic + 1)
                s0.wait()
                s1.wait()
                return 0

        assert nst % 2 == 0
        for half in range(nhalf):
            if half > 0:
                pltpu.sync_copy(dst_hbm.at[pl.ds(drow + half * nst, nst)],
                                dbig)
                if gather:
                    pltpu.sync_copy(src_hbm.at[pl.ds(srow + half * nst, nst)],
                                    sbig)
            lax.fori_loop(0, nst // 2, edge_body, 0)
        plsc.subcore_barrier()

        def wb(off, sz):
            pltpu.sync_copy(acc.at[pl.ds(off, sz)],
                            agg_hbm.at[pl.ds(c * N + off, sz)])

        _for_sub_rows(s, WBK, wb)

    if not gather:
        def body_nogather(src_hbm, dst_hbm, agg_hbm, *rest):
            return body(None, src_hbm, dst_hbm, agg_hbm, *rest)
        fn = body_nogather
    else:
        fn = body
    return pl.kernel(fn, out_type=jax.ShapeDtypeStruct((2 * N, 128), f32),
                     mesh=mesh, scratch_types=scratch)


_make_sc_agg = functools.lru_cache(maxsize=None)(_make_sc_agg)


def _sc_cols(tbl, src, dst):
    return _make_sc_agg("cols")(tbl, src, dst)


def _sc_rows(tbl, src, dst):
    return _make_sc_agg("rows")(tbl, src, dst)


def _sc_cnt(src, dst):
    return _make_sc_agg("cnt")(src, dst)


BR = 1000
NB = N // BR


def _tc_pre_body(x_ref, wl_ref, wr_ref, bl_ref, yl_ref, yr_ref):
    xb = x_ref[...]
    yl = lax.dot_general(xb, wl_ref[...], (((1,), (0,)), ((), ())),
                         preferred_element_type=f32)
    yl_ref[0] = yl[:, :128]
    yl_ref[1] = yl[:, 128:]
    yr_ref[...] = lax.dot_general(xb, wr_ref[...], (((1,), (0,)), ((), ())),
                                  preferred_element_type=f32) + bl_ref[0]


def _tc_pre(x, wl, wr, bl):
    return pl.pallas_call(
        _tc_pre_body,
        grid=(NB,),
        in_specs=[
            pl.BlockSpec((BR, DH), lambda i: (i, 0)),
            pl.BlockSpec((DH, DH), lambda i: (0, 0)),
            pl.BlockSpec((DH, DH), lambda i: (0, 0)),
            pl.BlockSpec((1, DH), lambda i: (0, 0)),
        ],
        out_specs=[
            pl.BlockSpec((2, BR, 128), lambda i: (0, i, 0)),
            pl.BlockSpec((BR, DH), lambda i: (i, 0)),
        ],
        out_shape=[
            jax.ShapeDtypeStruct((2, N, 128), f32),
            jax.ShapeDtypeStruct((N, DH), f32),
        ],
    )(x, wl, wr, bl[None, :])


def _tc_mid_body(split_out, dout,
                 agg_ref, cnt_ref, yr_ref, g_ref, be_ref, wl_ref, wr_ref,
                 bl_ref, yl_ref, yr2_ref, stats):
    p = pl.program_id(0)
    i = pl.program_id(1)
    agg = jnp.concatenate([agg_ref[0], agg_ref[1]], axis=1)
    cnt = jnp.maximum(cnt_ref[0] + cnt_ref[1], 1.0)
    h = agg / cnt + yr_ref[...]

    @pl.when(p == 0)
    def _():
        @pl.when(i == 0)
        def _():
            stats[...] = jnp.zeros_like(stats)
        stats[0:1, :] += jnp.sum(h, axis=0, keepdims=True)
        stats[1:2, :] += jnp.sum(h * h, axis=0, keepdims=True)

    @pl.when(p == 1)
    def _():
        mu = stats[0:1, :] / N
        var = stats[1:2, :] / N - mu * mu
        hn = (h - mu) * lax.rsqrt(var + EPS) * g_ref[0] + be_ref[0]
        hn = jnp.maximum(hn, 0.0)
        yl = lax.dot_general(hn, wl_ref[...], (((1,), (0,)), ((), ())),
                             preferred_element_type=f32)
        if split_out:
            yl_ref[0] = yl[:, :dout // 2]
            yl_ref[1] = yl[:, dout // 2:]
        else:
            yl_ref[...] = yl
        yr2_ref[...] = lax.dot_general(hn, wr_ref[...], (((1,), (0,)), ((), ())),
                                       preferred_element_type=f32) + bl_ref[0]


def _tc_mid(agg, cntp, yr, g, be, wl, wr, bl, split_out):
    dout = wl.shape[1]
    if split_out:
        yl_spec = pl.BlockSpec((2, BR, dout // 2), lambda p, i: (0, i, 0))
        yl_shape = jax.ShapeDtypeStruct((2, N, dout // 2), f32)
    else:
        yl_spec = pl.BlockSpec((BR, dout), lambda p, i: (i, 0))
        yl_shape = jax.ShapeDtypeStruct((N, dout), f32)
    return pl.pallas_call(
        functools.partial(_tc_mid_body, split_out, dout),
        grid=(2, NB),
        in_specs=[
            pl.BlockSpec((2, BR, 128), lambda p, i: (0, i, 0)),
            pl.BlockSpec((2, BR, 1), lambda p, i: (0, i, 0)),
            pl.BlockSpec((BR, DH), lambda p, i: (i, 0)),
            pl.BlockSpec((1, DH), lambda p, i: (0, 0)),
            pl.BlockSpec((1, DH), lambda p, i: (0, 0)),
            pl.BlockSpec((DH, dout), lambda p, i: (0, 0)),
            pl.BlockSpec((DH, dout), lambda p, i: (0, 0)),
            pl.BlockSpec((1, dout), lambda p, i: (0, 0)),
        ],
        out_specs=[
            yl_spec,
            pl.BlockSpec((BR, dout), lambda p, i: (i, 0)),
        ],
        out_shape=[
            yl_shape,
            jax.ShapeDtypeStruct((N, dout), f32),
        ],
        scratch_shapes=[pltpu.VMEM((8, DH), f32)],
    )(agg, cntp, yr, g[None, :], be[None, :], wl, wr, bl[None, :])


def _tc_post_body(part_ref, cnt_ref, yr_ref, out_ref):
    agg = part_ref[0] + part_ref[1]
    cnt = jnp.maximum(cnt_ref[0] + cnt_ref[1], 1.0)
    out_ref[...] = agg / cnt + yr_ref[...]


def _tc_post(part, cntp, yr):
    return pl.pallas_call(
        _tc_post_body,
        grid=(NB,),
        in_specs=[
            pl.BlockSpec((2, BR, DO), lambda i: (0, i, 0)),
            pl.BlockSpec((2, BR, 1), lambda i: (0, i, 0)),
            pl.BlockSpec((BR, DO), lambda i: (i, 0)),
        ],
        out_specs=pl.BlockSpec((BR, DO), lambda i: (i, 0)),
        out_shape=jax.ShapeDtypeStruct((N, DO), f32),
    )(part, cntp, yr)


def kernel(x, edge_index, Wl0, bl0, Wr0, g0, be0, Wl1, bl1, Wr1, g1, be1,
           Wl2, bl2, Wr2):
    src = edge_index[0]
    dst = edge_index[1]
    K = 125
    src2 = jnp.concatenate([src, src + N]).reshape(2 * E // K, K)
    srcr = src.reshape(E // K, K)
    dstr = dst.reshape(E // K, K)

    cntp = _sc_cnt(srcr, dstr).reshape(2, N, 128)[:, :, 0:1]
    yl0, yr0 = _tc_pre(x, Wl0, Wr0, bl0)
    agg0 = _sc_cols(yl0.reshape(2 * N, 128), src2, dstr)
    yl1, yr1 = _tc_mid(agg0.reshape(2, N, 128), cntp, yr0, g0, be0,
                       Wl1, Wr1, bl1, split_out=True)
    agg1 = _sc_cols(yl1.reshape(2 * N, 128), src2, dstr)
    yl2, yr2 = _tc_mid(agg1.reshape(2, N, 128), cntp, yr1, g1, be1,
                       Wl2, Wr2, bl2, split_out=False)
    part2 = _sc_rows(yl2, srcr, dstr)
    return _tc_post(part2.reshape(2, N, DO), cntp, yr2)


# speedup vs baseline: 8.7787x; 1.1237x over previous
import functools

import jax
import jax.numpy as jnp
from jax import lax
from jax.experimental import pallas as pl
from jax.experimental.pallas import tpu as pltpu
from jax.experimental.pallas import tpu_sc as plsc

N = 10000
E = 160000
DH = 256
DO = 128
EPS = 1e-5

NC = 2
NS = 16
RS = 624

f32 = jnp.float32
i32 = jnp.int32


def _zero_rows(ref, nrows, ncols, value=0.0):
    v16 = jnp.full((16,), value, f32)

    def body(r, _):
        for w in range(ncols // 16):
            ref[r, pl.ds(w * 16, 16)] = v16
        return 0

    lax.fori_loop(0, nrows, body, 0)


def _row_chunks(total, chunk):
    full = total // chunk
    out = [(i * chunk, chunk) for i in range(full)]
    if total - full * chunk:
        out.append((full * chunk, total - full * chunk))
    return out


def _for_sub_rows(s, chunk, fn):
    for off, sz in _row_chunks(RS, chunk):
        fn(s * RS + off, sz)

    @pl.when(s == NS - 1)
    def _():
        fn(NS * RS, N - NS * RS)


def _make_sc_agg(mode):
    col_split = mode == "cols"
    gather = mode != "cnt"
    K = 125
    if col_split:
        nch = (E // NS) // K
    else:
        nch = (E // (NC * NS)) // K
    WBK = 80

    nhalf = 2 if col_split else 1
    nst = nch // nhalf
    scratch = [
        pltpu.VMEM((nst, K), i32),
        pltpu.VMEM((K, 128), f32),
        pltpu.VMEM_SHARED((N, 128), f32),
        pltpu.SemaphoreType.DMA,
    ]
    if gather:
        scratch += [
            pltpu.VMEM((nst, K), i32),
            pltpu.VMEM((K, 128), f32),
            pltpu.SemaphoreType.DMA,
        ]

    mesh = plsc.VectorSubcoreMesh(core_axis_name="c", subcore_axis_name="s",
                                  num_cores=NC, num_subcores=NS)

    def body(tbl_hbm, src_hbm, dst_hbm, agg_hbm, dbig, rows0, acc, ssem,
             *rest):
        if gather:
            sbig, rows1, gsem = rest
            rows = [rows0, rows1]
        else:
            rows = [rows0, rows0]
        c = lax.axis_index("c")
        s = lax.axis_index("s")

        if col_split:
            drow = s * nch
            srow = c * (E // K) + s * nch
        else:
            drow = c * (E // (NC * K)) + s * nch
            srow = drow

        pltpu.sync_copy(dst_hbm.at[pl.ds(drow, nst)], dbig)
        if gather:
            pltpu.sync_copy(src_hbm.at[pl.ds(srow, nst)], sbig)

        _zero_rows(rows0, WBK, 128)
        _for_sub_rows(s, WBK, lambda off, sz: pltpu.sync_copy(
            rows0.at[pl.ds(0, sz)], acc.at[pl.ds(off, sz)]))
        if not gather:
            _zero_rows(rows0, K, 128, value=1.0)
        plsc.subcore_barrier()

        def issue_g(b, ic):
            return pltpu.async_copy(tbl_hbm.at[sbig.at[ic]], rows[b], gsem)

        def issue_s(b, ic):
            return pltpu.async_copy(rows[b], acc.at[dbig.at[ic]], ssem,
                                    add=True)

        UNROLL = 20
        assert nst % UNROLL == 0
        if gather:
            def edge_body(it, _):
                c0 = it * UNROLL
                g_cur = issue_g(0, c0)
                s_prev = None
                for j in range(UNROLL):
                    b = j % 2
                    if s_prev is not None:
                        s_prev.wait()
                    if j + 1 < UNROLL:
                        g_next = issue_g(1 - b, c0 + j + 1)
                    g_cur.wait()
                    s_prev = issue_s(b, c0 + j)
                    if j + 1 < UNROLL:
                        g_cur = g_next
                s_prev.wait()
                return 0
        else:
            def edge_body(it, _):
                c0 = it * UNROLL
                descs = [issue_s(0, c0 + j) for j in range(UNROLL)]
                for d in descs:
                    d.wait()
                return 0

        for half in range(nhalf):
            if half > 0:
                pltpu.sync_copy(dst_hbm.at[pl.ds(drow + half * nst, nst)],
                                dbig)
                if gather:
                    pltpu.sync_copy(src_hbm.at[pl.ds(srow + half * nst, nst)],
                                    sbig)
            lax.fori_loop(0, nst // UNROLL, edge_body, 0)
        plsc.subcore_barrier()

        def wb(off, sz):
            pltpu.sync_copy(acc.at[pl.ds(off, sz)],
                            agg_hbm.at[pl.ds(c * N + off, sz)])

        _for_sub_rows(s, WBK, wb)

    if not gather:
        def body_nogather(src_hbm, dst_hbm, agg_hbm, *rest):
            return body(None, src_hbm, dst_hbm, agg_hbm, *rest)
        fn = body_nogather
    else:
        fn = body
    return pl.kernel(fn, out_type=jax.ShapeDtypeStruct((2 * N, 128), f32),
                     mesh=mesh, scratch_types=scratch)


_make_sc_agg = functools.lru_cache(maxsize=None)(_make_sc_agg)


def _sc_cols(tbl, src, dst):
    return _make_sc_agg("cols")(tbl, src, dst)


def _sc_rows(tbl, src, dst):
    return _make_sc_agg("rows")(tbl, src, dst)


def _sc_cnt(src, dst):
    return _make_sc_agg("cnt")(src, dst)


BR = 1000
NB = N // BR


def _tc_pre_body(x_ref, wl_ref, wr_ref, bl_ref, yl_ref, yr_ref):
    xb = x_ref[...]
    yl = lax.dot_general(xb, wl_ref[...], (((1,), (0,)), ((), ())),
                         preferred_element_type=f32)
    yl_ref[0] = yl[:, :128]
    yl_ref[1] = yl[:, 128:]
    yr_ref[...] = lax.dot_general(xb, wr_ref[...], (((1,), (0,)), ((), ())),
                                  preferred_element_type=f32) + bl_ref[0]


def _tc_pre(x, wl, wr, bl):
    return pl.pallas_call(
        _tc_pre_body,
        grid=(NB,),
        in_specs=[
            pl.BlockSpec((BR, DH), lambda i: (i, 0)),
            pl.BlockSpec((DH, DH), lambda i: (0, 0)),
            pl.BlockSpec((DH, DH), lambda i: (0, 0)),
            pl.BlockSpec((1, DH), lambda i: (0, 0)),
        ],
        out_specs=[
            pl.BlockSpec((2, BR, 128), lambda i: (0, i, 0)),
            pl.BlockSpec((BR, DH), lambda i: (i, 0)),
        ],
        out_shape=[
            jax.ShapeDtypeStruct((2, N, 128), f32),
            jax.ShapeDtypeStruct((N, DH), f32),
        ],
    )(x, wl, wr, bl[None, :])


def _tc_mid_body(split_out, dout,
                 agg_ref, cnt_ref, yr_ref, g_ref, be_ref, wl_ref, wr_ref,
                 bl_ref, yl_ref, yr2_ref, stats):
    p = pl.program_id(0)
    i = pl.program_id(1)
    agg = jnp.concatenate([agg_ref[0], agg_ref[1]], axis=1)
    cnt = jnp.maximum(cnt_ref[0] + cnt_ref[1], 1.0)
    h = agg / cnt + yr_ref[...]

    @pl.when(p == 0)
    def _():
        @pl.when(i == 0)
        def _():
            stats[...] = jnp.zeros_like(stats)
        stats[0:1, :] += jnp.sum(h, axis=0, keepdims=True)
        stats[1:2, :] += jnp.sum(h * h, axis=0, keepdims=True)

    @pl.when(p == 1)
    def _():
        mu = stats[0:1, :] / N
        var = stats[1:2, :] / N - mu * mu
        hn = (h - mu) * lax.rsqrt(var + EPS) * g_ref[0] + be_ref[0]
        hn = jnp.maximum(hn, 0.0)
        yl = lax.dot_general(hn, wl_ref[...], (((1,), (0,)), ((), ())),
                             preferred_element_type=f32)
        if split_out:
            yl_ref[0] = yl[:, :dout // 2]
            yl_ref[1] = yl[:, dout // 2:]
        else:
            yl_ref[...] = yl
        yr2_ref[...] = lax.dot_general(hn, wr_ref[...], (((1,), (0,)), ((), ())),
                                       preferred_element_type=f32) + bl_ref[0]


def _tc_mid(agg, cntp, yr, g, be, wl, wr, bl, split_out):
    dout = wl.shape[1]
    if split_out:
        yl_spec = pl.BlockSpec((2, BR, dout // 2), lambda p, i: (0, i, 0))
        yl_shape = jax.ShapeDtypeStruct((2, N, dout // 2), f32)
    else:
        yl_spec = pl.BlockSpec((BR, dout), lambda p, i: (i, 0))
        yl_shape = jax.ShapeDtypeStruct((N, dout), f32)
    return pl.pallas_call(
        functools.partial(_tc_mid_body, split_out, dout),
        grid=(2, NB),
        in_specs=[
            pl.BlockSpec((2, BR, 128), lambda p, i: (0, i, 0)),
            pl.BlockSpec((2, BR, 1), lambda p, i: (0, i, 0)),
            pl.BlockSpec((BR, DH), lambda p, i: (i, 0)),
            pl.BlockSpec((1, DH), lambda p, i: (0, 0)),
            pl.BlockSpec((1, DH), lambda p, i: (0, 0)),
            pl.BlockSpec((DH, dout), lambda p, i: (0, 0)),
            pl.BlockSpec((DH, dout), lambda p, i: (0, 0)),
            pl.BlockSpec((1, dout), lambda p, i: (0, 0)),
        ],
        out_specs=[
            yl_spec,
            pl.BlockSpec((BR, dout), lambda p, i: (i, 0)),
        ],
        out_shape=[
            yl_shape,
            jax.ShapeDtypeStruct((N, dout), f32),
        ],
        scratch_shapes=[pltpu.VMEM((8, DH), f32)],
    )(agg, cntp, yr, g[None, :], be[None, :], wl, wr, bl[None, :])


def _tc_post_body(part_ref, cnt_ref, yr_ref, out_ref):
    agg = part_ref[0] + part_ref[1]
    cnt = jnp.maximum(cnt_ref[0] + cnt_ref[1], 1.0)
    out_ref[...] = agg / cnt + yr_ref[...]


def _tc_post(part, cntp, yr):
    return pl.pallas_call(
        _tc_post_body,
        grid=(NB,),
        in_specs=[
            pl.BlockSpec((2, BR, DO), lambda i: (0, i, 0)),
            pl.BlockSpec((2, BR, 1), lambda i: (0, i, 0)),
            pl.BlockSpec((BR, DO), lambda i: (i, 0)),
        ],
        out_specs=pl.BlockSpec((BR, DO), lambda i: (i, 0)),
        out_shape=jax.ShapeDtypeStruct((N, DO), f32),
    )(part, cntp, yr)


def kernel(x, edge_index, Wl0, bl0, Wr0, g0, be0, Wl1, bl1, Wr1, g1, be1,
           Wl2, bl2, Wr2):
    src = edge_index[0]
    dst = edge_index[1]
    K = 125
    src2 = jnp.concatenate([src, src + N]).reshape(2 * E // K, K)
    srcr = src.reshape(E // K, K)
    dstr = dst.reshape(E // K, K)

    cntp = _sc_cnt(srcr, dstr).reshape(2, N, 128)[:, :, 0:1]
    yl0, yr0 = _tc_pre(x, Wl0, Wr0, bl0)
    agg0 = _sc_cols(yl0.reshape(2 * N, 128), src2, dstr)
    yl1, yr1 = _tc_mid(agg0.reshape(2, N, 128), cntp, yr0, g0, be0,
                       Wl1, Wr1, bl1, split_out=True)
    agg1 = _sc_cols(yl1.reshape(2 * N, 128), src2, dstr)
    yl2, yr2 = _tc_mid(agg1.reshape(2, N, 128), cntp, yr1, g1, be1,
                       Wl2, Wr2, bl2, split_out=False)
    part2 = _sc_rows(yl2, srcr, dstr)
    return _tc_post(part2.reshape(2, N, DO), cntp, yr2)
